# baseline (speedup 1.0000x reference)
import jax
import jax.numpy as jnp
from jax import lax
from jax.experimental import pallas as pl
from jax.experimental.pallas import tpu as pltpu, tpu_sc as plsc

B, S = 4096, 26
DIM = 64
NC, NS = 2, 16
NW = NC * NS
BPW = B // NW
CX = 16
NCHUNK = BPW // CX
NB = 2
LANES = 16


def _emb_body(x_hbm, table_hbm, p_hbm, idx_v, rows_v, trows_v,
              gs0, gs1, ws0, ws1):
    gs = (gs0, gs1)
    ws = (ws0, ws1)
    wid = lax.axis_index("s") * NC + lax.axis_index("c")
    base = wid * BPW
    pltpu.sync_copy(x_hbm.at[pl.ds(base, BPW)], idx_v)

    lane = lax.iota(jnp.int32, LANES)
    khi = [lax.shift_right_arithmetic(k16 * LANES + lane, 3)
           for k16 in range(DIM // LANES)]
    klo = [lax.bitwise_and(k16 * LANES + lane, 7)
           for k16 in range(DIM // LANES)]
    isplat = [lax.full((LANES,), i, jnp.int32) for i in range(CX)]

    def fire_gathers(c, nb):
        return [
            pltpu.async_copy(
                table_hbm.at[idx_v.at[c * CX + i]], rows_v.at[nb, i], gs[nb])
            for i in range(CX)
        ]

    def transpose_chunk(nb):
        def tbody(j, carry):
            jv = lax.full((LANES,), 0, jnp.int32) + j
            for i in range(CX):
                for k16 in range(DIM // LANES):
                    v = rows_v[nb, i, j, pl.ds(k16 * LANES, LANES)]
                    plsc.store_scatter(
                        trows_v.at[nb], [jv, khi[k16], klo[k16], isplat[i]], v)
            return carry
        lax.fori_loop(0, S, tbody, 0)

    gd = {0: fire_gathers(0, 0)}
    wd = {}
    for c in range(NCHUNK):
        nb = c % NB
        if c + 1 < NCHUNK:
            gd[c + 1] = fire_gathers(c + 1, 1 - nb)
        if c >= NB:
            wd[c - NB].wait()
        for d in gd[c]:
            d.wait()
        transpose_chunk(nb)
        wd[c] = pltpu.async_copy(
            trows_v.at[nb],
            p_hbm.at[:, :, wid, :, pl.ds(c * CX, CX)],
            ws[nb])
    for c in range(NCHUNK - NB, NCHUNK):
        wd[c].wait()


@jax.jit
def _embedding_sc(x, weight):
    mesh = plsc.VectorSubcoreMesh(core_axis_name="c", subcore_axis_name="s")
    f = pl.kernel(
        _emb_body,
        out_type=jax.ShapeDtypeStruct((S, DIM // 8, NW, 8, 128), jnp.float32),
        mesh=mesh,
        scratch_types=[
            pltpu.VMEM((BPW, S), jnp.int32),
            pltpu.VMEM((NB, CX, S, DIM), jnp.float32),
            pltpu.VMEM((NB, S, DIM // 8, 8, CX), jnp.float32),
            pltpu.SemaphoreType.DMA,
            pltpu.SemaphoreType.DMA,
            pltpu.SemaphoreType.DMA,
            pltpu.SemaphoreType.DMA,
        ],
        compiler_params=pltpu.CompilerParams(
            use_tc_tiling_on_sc=False, needs_layout_passes=False),
    )
    p = f(x, weight)
    return jnp.transpose(p, (2, 4, 0, 1, 3)).reshape(B, S, DIM)


def kernel(x, weight):
    return _embedding_sc(x, weight)


# speedup vs baseline: 1.0805x; 1.0805x over previous
import jax
import jax.numpy as jnp
from jax import lax
from jax.experimental import pallas as pl
from jax.experimental.pallas import tpu as pltpu, tpu_sc as plsc

B, S = 4096, 26
DIM = 64
NC, NS = 2, 16
NW = NC * NS
BPW = B // NW
CX = 16
NCHUNK = BPW // CX
NB = 2
LANES = 16


def _emb_body(x_hbm, table_hbm, p_hbm, idx_v, rows_v, trows_v,
              gs0, gs1, ws0):
    gs = (gs0, gs1)
    wid = lax.axis_index("s") * NC + lax.axis_index("c")
    base = wid * BPW
    pltpu.sync_copy(x_hbm.at[pl.ds(base, BPW)], idx_v)

    lane = lax.iota(jnp.int32, LANES)
    khi = [lax.shift_right_arithmetic(k16 * LANES + lane, 3)
           for k16 in range(DIM // LANES)]
    klo = [lax.bitwise_and(k16 * LANES + lane, 7)
           for k16 in range(DIM // LANES)]
    isplat = [lax.full((LANES,), i, jnp.int32) for i in range(CX)]

    def fire_gathers(c, nb):
        return [
            pltpu.async_copy(
                table_hbm.at[idx_v.at[c * CX + i]], rows_v.at[nb, i], gs[nb])
            for i in range(CX)
        ]

    def transpose_chunk(nb):
        def tbody(j, carry):
            jv = lax.full((LANES,), 0, jnp.int32) + j
            for i in range(CX):
                for k16 in range(DIM // LANES):
                    v = rows_v[nb, i, j, pl.ds(k16 * LANES, LANES)]
                    plsc.store_scatter(
                        trows_v, [jv, khi[k16], klo[k16], isplat[i]], v)
            return carry
        lax.fori_loop(0, S, tbody, 0)

    gd = {0: fire_gathers(0, 0)}
    wd = {}
    for c in range(NCHUNK):
        nb = c % NB
        if c + 1 < NCHUNK:
            gd[c + 1] = fire_gathers(c + 1, 1 - nb)
        for d in gd[c]:
            d.wait()
        if c >= 1:
            wd[c - 1].wait()
        transpose_chunk(nb)
        wd[c] = pltpu.async_copy(
            trows_v.at[:, :, :, pl.ds(0, CX)],
            p_hbm.at[:, :, wid, :, pl.ds(c * CX, CX)],
            ws0)
    wd[NCHUNK - 1].wait()


@jax.jit
def _embedding_sc(x, weight):
    mesh = plsc.VectorSubcoreMesh(core_axis_name="c", subcore_axis_name="s")
    f = pl.kernel(
        _emb_body,
        out_type=jax.ShapeDtypeStruct((S, DIM // 8, NW, 8, 128), jnp.float32),
        mesh=mesh,
        scratch_types=[
            pltpu.VMEM((BPW, S), jnp.int32),
            pltpu.VMEM((NB, CX, S, DIM), jnp.float32),
            pltpu.VMEM((S, DIM // 8, 8, CX + 1), jnp.float32),
            pltpu.SemaphoreType.DMA,
            pltpu.SemaphoreType.DMA,
            pltpu.SemaphoreType.DMA,
        ],
        compiler_params=pltpu.CompilerParams(
            use_tc_tiling_on_sc=False, needs_layout_passes=False),
    )
    p = f(x, weight)
    return jnp.transpose(p, (2, 4, 0, 1, 3)).reshape(B, S, DIM)


def kernel(x, weight):
    return _embedding_sc(x, weight)
